# baseline (speedup 1.0000x reference)
import jax
import jax.numpy as jnp
from jax.experimental import pallas as pl
from jax.experimental.pallas import tpu as pltpu

_D_MODEL = 768
_N_EXP = 64
_TOPK = 8
_BT = 8192


def _gate_body(x_ref, w_ref, b_ref, idx_ref, score_ref):
    x = x_ref[...]
    w = w_ref[...]
    gate = jax.lax.dot_general(
        w, x, (((1,), (1,)), ((), ())), preferred_element_type=jnp.float32
    )
    gate = gate + b_ref[...]

    iota = jax.lax.broadcasted_iota(jnp.int32, gate.shape, 0)
    vals = gate
    top_vals, top_idx = [], []
    for k in range(_TOPK):
        m = jnp.max(vals, axis=0, keepdims=True)
        eq = vals == m
        amin = jnp.min(jnp.where(eq, iota, _N_EXP), axis=0, keepdims=True)
        top_vals.append(m)
        top_idx.append(amin)
        if k < _TOPK - 1:
            vals = jnp.where(iota == amin, -jnp.inf, vals)

    tv = jnp.concatenate(top_vals, axis=0)
    ti = jnp.concatenate(top_idx, axis=0)
    e = jnp.exp(tv - tv[:1])
    score_ref[...] = e / jnp.sum(e, axis=0, keepdims=True)
    idx_ref[...] = ti


def kernel(x, W, b):
    T = x.shape[0]
    b2 = b.reshape(_N_EXP, 1)
    idx_t, scores_t = pl.pallas_call(
        _gate_body,
        grid=(T // _BT,),
        in_specs=[
            pl.BlockSpec((_BT, _D_MODEL), lambda i: (i, 0)),
            pl.BlockSpec((_N_EXP, _D_MODEL), lambda i: (0, 0)),
            pl.BlockSpec((_N_EXP, 1), lambda i: (0, 0)),
        ],
        out_specs=[
            pl.BlockSpec((_TOPK, _BT), lambda i: (0, i)),
            pl.BlockSpec((_TOPK, _BT), lambda i: (0, i)),
        ],
        out_shape=[
            jax.ShapeDtypeStruct((_TOPK, T), jnp.int32),
            jax.ShapeDtypeStruct((_TOPK, T), jnp.float32),
        ],
        compiler_params=pltpu.CompilerParams(
            dimension_semantics=("arbitrary",),
        ),
    )(x, W, b2)
    return idx_t.T.astype(jnp.int64), scores_t.T


# speedup vs baseline: 1.0597x; 1.0597x over previous
import jax
import jax.numpy as jnp
from jax.experimental import pallas as pl
from jax.experimental.pallas import tpu as pltpu

_D_MODEL = 768
_N_EXP = 64
_TOPK = 8
_BT = 4096


def _gate_body(x_ref, w_ref, b_ref, idx_ref, score_ref):
    x = x_ref[...]
    w = w_ref[...]
    gate = jax.lax.dot_general(
        w, x, (((1,), (1,)), ((), ())), preferred_element_type=jnp.float32
    )
    gate = gate + b_ref[...]

    fiota = jax.lax.broadcasted_iota(jnp.int32, gate.shape, 0).astype(jnp.float32)
    vals = gate
    top_vals, top_idx = [], []
    for k in range(_TOPK):
        m = jnp.max(vals, axis=0, keepdims=True)
        eq = vals == m
        amin = jnp.min(jnp.where(eq, fiota, 64.0), axis=0, keepdims=True)
        top_vals.append(m)
        top_idx.append(amin)
        if k < _TOPK - 1:
            vals = jnp.where(eq, -jnp.inf, vals)

    tv = jnp.concatenate(top_vals, axis=0)
    ti = jnp.concatenate(top_idx, axis=0).astype(jnp.int32)
    e = jnp.exp(tv - tv[:1])
    score_ref[...] = e / jnp.sum(e, axis=0, keepdims=True)
    idx_ref[...] = ti


def kernel(x, W, b):
    T = x.shape[0]
    b2 = b.reshape(_N_EXP, 1)
    idx_t, scores_t = pl.pallas_call(
        _gate_body,
        grid=(T // _BT,),
        in_specs=[
            pl.BlockSpec((_BT, _D_MODEL), lambda i: (i, 0)),
            pl.BlockSpec((_N_EXP, _D_MODEL), lambda i: (0, 0)),
            pl.BlockSpec((_N_EXP, 1), lambda i: (0, 0)),
        ],
        out_specs=[
            pl.BlockSpec((_TOPK, _BT), lambda i: (0, i)),
            pl.BlockSpec((_TOPK, _BT), lambda i: (0, i)),
        ],
        out_shape=[
            jax.ShapeDtypeStruct((_TOPK, T), jnp.int32),
            jax.ShapeDtypeStruct((_TOPK, T), jnp.float32),
        ],
        compiler_params=pltpu.CompilerParams(
            dimension_semantics=("arbitrary",),
        ),
    )(x, W, b2)
    return idx_t.T.astype(jnp.int64), scores_t.T


# speedup vs baseline: 1.0681x; 1.0080x over previous
import jax
import jax.numpy as jnp
from jax.experimental import pallas as pl
from jax.experimental.pallas import tpu as pltpu

_D_MODEL = 768
_N_EXP = 64
_TOPK = 8
_BT = 4096


def _gate_body(x_ref, w_ref, b_ref, idx_ref, score_ref):
    x = x_ref[...]
    w = w_ref[...]
    gate = jax.lax.dot_general(
        w, x, (((1,), (1,)), ((), ())), preferred_element_type=jnp.float32
    )
    gate = gate + b_ref[...]

    fiota = jax.lax.broadcasted_iota(jnp.int32, gate.shape, 0).astype(jnp.float32)
    vals = gate
    top_vals, top_idx = [], []
    for k in range(_TOPK):
        m = jnp.max(vals, axis=0, keepdims=True)
        eq = vals == m
        amin = jnp.min(jnp.where(eq, fiota, 64.0), axis=0, keepdims=True)
        top_vals.append(m)
        top_idx.append(amin)
        if k < _TOPK - 1:
            vals = jnp.where(fiota == amin, -jnp.inf, vals)

    tv = jnp.concatenate(top_vals, axis=0)
    ti = jnp.concatenate(top_idx, axis=0).astype(jnp.int32)
    e = jnp.exp(tv - tv[:1])
    score_ref[...] = e / jnp.sum(e, axis=0, keepdims=True)
    idx_ref[...] = ti


def kernel(x, W, b):
    T = x.shape[0]
    b2 = b.reshape(_N_EXP, 1)
    idx_t, scores_t = pl.pallas_call(
        _gate_body,
        grid=(T // _BT,),
        in_specs=[
            pl.BlockSpec((_BT, _D_MODEL), lambda i: (i, 0)),
            pl.BlockSpec((_N_EXP, _D_MODEL), lambda i: (0, 0)),
            pl.BlockSpec((_N_EXP, 1), lambda i: (0, 0)),
        ],
        out_specs=[
            pl.BlockSpec((_TOPK, _BT), lambda i: (0, i)),
            pl.BlockSpec((_TOPK, _BT), lambda i: (0, i)),
        ],
        out_shape=[
            jax.ShapeDtypeStruct((_TOPK, T), jnp.int32),
            jax.ShapeDtypeStruct((_TOPK, T), jnp.float32),
        ],
        compiler_params=pltpu.CompilerParams(
            dimension_semantics=("arbitrary",),
        ),
    )(x, W, b2)
    return idx_t.T.astype(jnp.int64), scores_t.T
